# baseline (speedup 1.0000x reference)
import jax
import jax.numpy as jnp
from jax.experimental import pallas as pl

_BM1 = 2000
_BM = 400


def _support_body(x_ref, w1_ref, s_ref):
    xb = x_ref[...].astype(jnp.bfloat16)
    s = jax.lax.dot_general(xb, w1_ref[...], (((1,), (0,)), ((), ())),
                            preferred_element_type=jnp.float32)
    s_ref[...] = s.astype(jnp.bfloat16)


def _layer1_body(adj_ref, s_ref, w2_ref, b1_ref, t_ref):
    a = adj_ref[...].astype(jnp.bfloat16)
    acc = jax.lax.dot_general(a, s_ref[...], (((1,), (0,)), ((), ())),
                              preferred_element_type=jnp.float32)
    h = jnp.maximum(acc + b1_ref[...], 0.0).astype(jnp.bfloat16)
    t = jax.lax.dot_general(h, w2_ref[...], (((1,), (0,)), ((), ())),
                            preferred_element_type=jnp.float32)
    t_ref[...] = t.astype(jnp.bfloat16)


def _layer2_body(adj_ref, t_ref, b2_ref, o_ref):
    a = adj_ref[...].astype(jnp.bfloat16)
    acc = jax.lax.dot_general(a, t_ref[...], (((1,), (0,)), ((), ())),
                              preferred_element_type=jnp.float32)
    o_ref[...] = jnp.maximum(acc + b2_ref[...], 0.0)


def kernel(x, adj, W1, b1, W2, b2):
    n, nfeat = x.shape
    nhid = W1.shape[1]
    nout = W2.shape[1]
    w1 = W1.astype(jnp.bfloat16)
    w2 = W2.astype(jnp.bfloat16)
    b1r = b1.reshape(1, nhid)
    b2r = b2.reshape(1, nout)

    support = pl.pallas_call(
        _support_body,
        grid=(n // _BM1,),
        in_specs=[
            pl.BlockSpec((_BM1, nfeat), lambda i: (i, 0)),
            pl.BlockSpec((nfeat, nhid), lambda i: (0, 0)),
        ],
        out_specs=pl.BlockSpec((_BM1, nhid), lambda i: (i, 0)),
        out_shape=jax.ShapeDtypeStruct((n, nhid), jnp.bfloat16),
    )(x, w1)

    t = pl.pallas_call(
        _layer1_body,
        grid=(n // _BM,),
        in_specs=[
            pl.BlockSpec((_BM, n), lambda i: (i, 0)),
            pl.BlockSpec((n, nhid), lambda i: (0, 0)),
            pl.BlockSpec((nhid, nout), lambda i: (0, 0)),
            pl.BlockSpec((1, nhid), lambda i: (0, 0)),
        ],
        out_specs=pl.BlockSpec((_BM, nout), lambda i: (i, 0)),
        out_shape=jax.ShapeDtypeStruct((n, nout), jnp.bfloat16),
    )(adj, support, w2, b1r)

    out = pl.pallas_call(
        _layer2_body,
        grid=(n // _BM,),
        in_specs=[
            pl.BlockSpec((_BM, n), lambda i: (i, 0)),
            pl.BlockSpec((n, nout), lambda i: (0, 0)),
            pl.BlockSpec((1, nout), lambda i: (0, 0)),
        ],
        out_specs=pl.BlockSpec((_BM, nout), lambda i: (i, 0)),
        out_shape=jax.ShapeDtypeStruct((n, nout), jnp.float32),
    )(adj, t, b2r)
    return out
